# baseline (speedup 1.0000x reference)
import jax
import jax.numpy as jnp
from jax import lax
from jax.experimental import pallas as pl
from jax.experimental.pallas import tpu as pltpu

DIM = 192
E = 8
H = 64
W = 64
HW = H * W


def _shift(a, dh, dw):
    if dh > 0:
        a = jnp.concatenate([a[dh:], jnp.zeros((dh,) + a.shape[1:], a.dtype)], axis=0)
    elif dh < 0:
        a = jnp.concatenate([jnp.zeros((-dh,) + a.shape[1:], a.dtype), a[:dh]], axis=0)
    if dw > 0:
        a = jnp.concatenate([a[:, dw:], jnp.zeros(a.shape[:1] + (dw,) + a.shape[2:], a.dtype)], axis=1)
    elif dw < 0:
        a = jnp.concatenate([jnp.zeros(a.shape[:1] + (-dw,) + a.shape[2:], a.dtype), a[:, :dw]], axis=1)
    return a


def _matmul_ct(a, w):
    return lax.dot_general(a, w, (((1,), (1,)), ((), ())),
                           preferred_element_type=jnp.float32)


def _moe_step(x_ref, rw_ref, rb_ref, eb_ref,
              w0_ref, b0_ref, dw0_ref, db0_ref,
              w1_ref, b1_ref, g1_ref,
              w2_ref, b2_ref,
              f3a_ref, f3ab_ref, f3b_ref, f3bb_ref,
              w4_ref, b4_ref, dw4_ref, db4_ref,
              w5_ref, b5_ref, g5_ref,
              w6_ref, b6_ref,
              f7a_ref, f7ab_ref, f7b_ref, f7bb_ref,
              out_ref, acc_ref):
    xn = x_ref[0]
    xb = jnp.transpose(xn, (1, 2, 0))
    xf = xb.reshape(HW, DIM)

    gvec = jnp.mean(xf, axis=0, keepdims=True)
    logits = _matmul_ct(gvec, rw_ref[...]) + rb_ref[...]
    logits = jnp.clip(logits, -10.0, 10.0) + eb_ref[...]
    m = jnp.max(logits)
    p = jnp.exp(logits - m)
    probs = p / jnp.sum(p)
    probs = jnp.clip(probs, 1e-6, 1.0)

    iota = lax.broadcasted_iota(jnp.int32, (1, E), 1)
    v1 = jnp.max(probs)
    i1 = jnp.min(jnp.where(probs == v1, iota, E))
    sel1 = iota == i1
    rest = jnp.where(sel1, -jnp.inf, probs)
    v2 = jnp.max(rest)
    i2 = jnp.min(jnp.where((rest == v2) & (~sel1), iota, E))
    sel2 = iota == i2
    denom = v1 + v2 + 1e-8
    wa = v1 / denom
    wb = v2 / denom
    gates = jnp.where(sel1, wa, 0.0) + jnp.where(sel2, wb, 0.0)

    def gate(e):
        return jnp.sum(jnp.where(iota == e, gates, 0.0))

    g0, g1, g2, g3 = gate(0), gate(1), gate(2), gate(3)
    g4, g5, g6, g7 = gate(4), gate(5), gate(6), gate(7)

    def s_vec(fa, fab, fb, fbb):
        h = jnp.maximum(_matmul_ct(gvec, fa[...]) + fab[...], 0.0)
        return jax.nn.sigmoid(_matmul_ct(h, fb[...]) + fbb[...])

    alpha = ((wa + wb)
             + g3 * s_vec(f3a_ref, f3ab_ref, f3b_ref, f3bb_ref)
             + g7 * s_vec(f7a_ref, f7ab_ref, f7b_ref, f7bb_ref))

    btot = (g0 * b0_ref[...] + g1 * b1_ref[...] + g2 * b2_ref[...]
            + g4 * b4_ref[...] + g5 * b5_ref[...] + g6 * b6_ref[...])

    acc_ref[...] = xf * alpha + btot

    @pl.when(g1 + g5 > 0.0)
    def _freq():
        wfr = g1 * (w1_ref[...] * g1_ref[...]) + g5 * (w5_ref[...] * g5_ref[...])
        acc_ref[...] += _matmul_ct(xf, wfr)

    @pl.when(g2 + g6 > 0.0)
    def _edge():
        lap = (_shift(xb, -1, 0) + _shift(xb, 1, 0) +
               _shift(xb, 0, -1) + _shift(xb, 0, 1) - 4.0 * xb)
        wed = g2 * w2_ref[...] + g6 * w6_ref[...]
        acc_ref[...] += _matmul_ct(lap.reshape(HW, DIM), wed)

    def texture(dw_ref, db_ref, w_ref, g):
        acc = jnp.broadcast_to(db_ref[...][None], (H, W, DIM))
        for a in range(3):
            for c in range(3):
                acc = acc + _shift(xb, a - 1, c - 1) * dw_ref[a * 3 + c][None, None, :]
        u = jax.nn.gelu(acc)
        acc_ref[...] += _matmul_ct(u.reshape(HW, DIM), g * w_ref[...])

    @pl.when(g0 > 0.0)
    def _tex0():
        texture(dw0_ref, db0_ref, w0_ref, g0)

    @pl.when(g4 > 0.0)
    def _tex4():
        texture(dw4_ref, db4_ref, w4_ref, g4)

    out_ref[0] = jnp.transpose(acc_ref[...].reshape(H, W, DIM), (2, 0, 1))


def kernel(x, params):
    B = x.shape[0]
    xh = x

    def pw(e):
        return params[f'e{e}_pw_w'].reshape(DIM, DIM)

    def row(v):
        return v[None, :]

    dw0 = params['e0_dw_w'].reshape(DIM, 9).T
    dw4 = params['e4_dw_w'].reshape(DIM, 9).T

    operands = [
        xh,
        params['router_w'], row(params['router_b']), row(params['expert_bias']),
        pw(0), row(params['e0_pw_b']), dw0, row(params['e0_dw_b']),
        pw(1), row(params['e1_pw_b']), row(params['e1_gain']),
        pw(2), row(params['e2_pw_b']),
        params['e3_fc1_w'], row(params['e3_fc1_b']),
        params['e3_fc2_w'], row(params['e3_fc2_b']),
        pw(4), row(params['e4_pw_b']), dw4, row(params['e4_dw_b']),
        pw(5), row(params['e5_pw_b']), row(params['e5_gain']),
        pw(6), row(params['e6_pw_b']),
        params['e7_fc1_w'], row(params['e7_fc1_b']),
        params['e7_fc2_w'], row(params['e7_fc2_b']),
    ]

    full = lambda a: pl.BlockSpec(a.shape, lambda b: (0,) * a.ndim)
    in_specs = [pl.BlockSpec((1, DIM, H, W), lambda b: (b, 0, 0, 0))]
    in_specs += [full(a) for a in operands[1:]]

    out_h = pl.pallas_call(
        _moe_step,
        grid=(B,),
        in_specs=in_specs,
        out_specs=pl.BlockSpec((1, DIM, H, W), lambda b: (b, 0, 0, 0)),
        out_shape=jax.ShapeDtypeStruct((B, DIM, H, W), jnp.float32),
        scratch_shapes=[pltpu.VMEM((HW, DIM), jnp.float32)],
        compiler_params=pltpu.CompilerParams(
            dimension_semantics=("parallel",)),
    )(*operands)

    return (out_h, jnp.array(0.0, dtype=x.dtype))


# speedup vs baseline: 1.6789x; 1.6789x over previous
import jax
import jax.numpy as jnp
from jax import lax
from jax.experimental import pallas as pl

DIM = 192
E = 8
H = 64
W = 64
HW = H * W

_OFF_RW = 0
_OFF_RB = 8
_OFF_EB = 9
_PW_ES = (0, 1, 2, 4, 5, 6)
_OFF_PW = {e: 10 + i * DIM for i, e in enumerate(_PW_ES)}
_OFF_PB = {e: 10 + 6 * DIM + i for i, e in enumerate(_PW_ES)}
_OFF_G1 = 10 + 6 * DIM + 6
_OFF_G5 = _OFF_G1 + 1
_OFF_DW0 = _OFF_G5 + 1
_OFF_DB0 = _OFF_DW0 + 9
_OFF_DW4 = _OFF_DB0 + 1
_OFF_DB4 = _OFF_DW4 + 9
_OFF_F3A = _OFF_DB4 + 1
_OFF_F3AB = _OFF_F3A + 48
_OFF_F3B = _OFF_F3AB + 1
_OFF_F3BB = _OFF_F3B + 48
_OFF_F7A = _OFF_F3BB + 1
_OFF_F7AB = _OFF_F7A + 48
_OFF_F7B = _OFF_F7AB + 1
_OFF_F7BB = _OFF_F7B + 48
_N_ROWS = _OFF_F7BB + 1


def _shift(a, dh, dw):
    if dh > 0:
        a = jnp.concatenate([a[dh:], jnp.zeros((dh,) + a.shape[1:], a.dtype)], axis=0)
    elif dh < 0:
        a = jnp.concatenate([jnp.zeros((-dh,) + a.shape[1:], a.dtype), a[:dh]], axis=0)
    if dw > 0:
        a = jnp.concatenate([a[:, dw:], jnp.zeros(a.shape[:1] + (dw,) + a.shape[2:], a.dtype)], axis=1)
    elif dw < 0:
        a = jnp.concatenate([jnp.zeros(a.shape[:1] + (-dw,) + a.shape[2:], a.dtype), a[:, :dw]], axis=1)
    return a


def _matmul_ct(a, w):
    return lax.dot_general(a, w, (((1,), (1,)), ((), ())),
                           preferred_element_type=jnp.float32)


def _matmul_nt(a, w):
    return lax.dot_general(a, w, (((1,), (0,)), ((), ())),
                           preferred_element_type=jnp.float32)


def _moe_step(x_ref, wt_ref, out_ref):
    xb = x_ref[0]
    xf = xb.reshape(HW, DIM)

    def rows(off, n):
        return wt_ref[off:off + n, :]

    gvec = jnp.mean(xf, axis=0, keepdims=True)
    logits = _matmul_ct(gvec, rows(_OFF_RW, E)) + wt_ref[_OFF_RB:_OFF_RB + 1, :E]
    logits = jnp.clip(logits, -10.0, 10.0) + wt_ref[_OFF_EB:_OFF_EB + 1, :E]
    m = jnp.max(logits)
    p = jnp.exp(logits - m)
    probs = p / jnp.sum(p)
    probs = jnp.clip(probs, 1e-6, 1.0)

    iota = lax.broadcasted_iota(jnp.int32, (1, E), 1)
    v1 = jnp.max(probs)
    i1 = jnp.min(jnp.where(probs == v1, iota, E))
    sel1 = iota == i1
    rest = jnp.where(sel1, -jnp.inf, probs)
    v2 = jnp.max(rest)
    i2 = jnp.min(jnp.where((rest == v2) & (~sel1), iota, E))
    sel2 = iota == i2
    denom = v1 + v2 + 1e-8
    wa = v1 / denom
    wb = v2 / denom
    gates = jnp.where(sel1, wa, 0.0) + jnp.where(sel2, wb, 0.0)

    def gate(e):
        return jnp.sum(jnp.where(iota == e, gates, 0.0))

    g0, g1, g2, g3 = gate(0), gate(1), gate(2), gate(3)
    g4, g5, g6, g7 = gate(4), gate(5), gate(6), gate(7)

    def s_vec(fa, fab, fb, fbb):
        h = jnp.maximum(_matmul_ct(gvec, rows(fa, 48)) + wt_ref[fab:fab + 1, :48], 0.0)
        return jax.nn.sigmoid(_matmul_nt(h, rows(fb, 48)) + wt_ref[fbb:fbb + 1, :])

    alpha = ((wa + wb)
             + g3 * s_vec(_OFF_F3A, _OFF_F3AB, _OFF_F3B, _OFF_F3BB)
             + g7 * s_vec(_OFF_F7A, _OFF_F7AB, _OFF_F7B, _OFF_F7BB))

    def brow(e):
        return wt_ref[_OFF_PB[e]:_OFF_PB[e] + 1, :]

    btot = (g0 * brow(0) + g1 * brow(1) + g2 * brow(2)
            + g4 * brow(4) + g5 * brow(5) + g6 * brow(6))

    out_ref[0] = (xf * alpha + btot).reshape(H, W, DIM)

    @pl.when(g1 + g5 > 0.0)
    def _freq():
        wfr = (g1 * (rows(_OFF_PW[1], DIM) * wt_ref[_OFF_G1:_OFF_G1 + 1, :])
               + g5 * (rows(_OFF_PW[5], DIM) * wt_ref[_OFF_G5:_OFF_G5 + 1, :]))
        out_ref[0] += _matmul_ct(xf, wfr).reshape(H, W, DIM)

    @pl.when(g2 + g6 > 0.0)
    def _edge():
        lap = (_shift(xb, -1, 0) + _shift(xb, 1, 0) +
               _shift(xb, 0, -1) + _shift(xb, 0, 1) - 4.0 * xb)
        wed = g2 * rows(_OFF_PW[2], DIM) + g6 * rows(_OFF_PW[6], DIM)
        out_ref[0] += _matmul_ct(lap.reshape(HW, DIM), wed).reshape(H, W, DIM)

    def texture(dw_off, db_off, w_off, g):
        acc = jnp.broadcast_to(wt_ref[db_off:db_off + 1, :][None], (H, W, DIM))
        for a in range(3):
            for c in range(3):
                acc = acc + _shift(xb, a - 1, c - 1) * wt_ref[dw_off + a * 3 + c][None, None, :]
        u = jax.nn.gelu(acc)
        out_ref[0] += _matmul_ct(u.reshape(HW, DIM), g * rows(w_off, DIM)).reshape(H, W, DIM)

    @pl.when(g0 > 0.0)
    def _tex0():
        texture(_OFF_DW0, _OFF_DB0, _OFF_PW[0], g0)

    @pl.when(g4 > 0.0)
    def _tex4():
        texture(_OFF_DW4, _OFF_DB4, _OFF_PW[4], g4)


def kernel(x, params):
    B = x.shape[0]
    xh = jnp.transpose(x, (0, 2, 3, 1))

    def padrow(v):
        return jnp.pad(v[None, :], ((0, 0), (0, DIM - v.shape[0])))

    def row(v):
        return v[None, :]

    pieces = [
        params['router_w'],
        padrow(params['router_b']),
        padrow(params['expert_bias']),
    ]
    for e in _PW_ES:
        pieces.append(params[f'e{e}_pw_w'].reshape(DIM, DIM))
    for e in _PW_ES:
        pieces.append(row(params[f'e{e}_pw_b']))
    pieces += [
        row(params['e1_gain']), row(params['e5_gain']),
        params['e0_dw_w'].reshape(DIM, 9).T, row(params['e0_dw_b']),
        params['e4_dw_w'].reshape(DIM, 9).T, row(params['e4_dw_b']),
        params['e3_fc1_w'], padrow(params['e3_fc1_b']),
        params['e3_fc2_w'].T, row(params['e3_fc2_b']),
        params['e7_fc1_w'], padrow(params['e7_fc1_b']),
        params['e7_fc2_w'].T, row(params['e7_fc2_b']),
    ]
    wt = jnp.concatenate(pieces, axis=0)
    assert wt.shape == (_N_ROWS, DIM), wt.shape

    out_h = pl.pallas_call(
        _moe_step,
        grid=(B,),
        in_specs=[
            pl.BlockSpec((1, H, W, DIM), lambda b: (b, 0, 0, 0)),
            pl.BlockSpec((_N_ROWS, DIM), lambda b: (0, 0)),
        ],
        out_specs=pl.BlockSpec((1, H, W, DIM), lambda b: (b, 0, 0, 0)),
        out_shape=jax.ShapeDtypeStruct((B, H, W, DIM), jnp.float32),
    )(xh, wt)

    out = jnp.transpose(out_h, (0, 3, 1, 2))
    return (out, jnp.array(0.0, dtype=x.dtype))


# speedup vs baseline: 2.7894x; 1.6614x over previous
import jax
import jax.numpy as jnp
from jax import lax
from jax.experimental import pallas as pl

DIM = 192
E = 8
H = 64
W = 64
HW = H * W


def _shift(a, dh, dw):
    if dh > 0:
        a = jnp.concatenate([a[dh:], jnp.zeros((dh,) + a.shape[1:], a.dtype)], axis=0)
    elif dh < 0:
        a = jnp.concatenate([jnp.zeros((-dh,) + a.shape[1:], a.dtype), a[:dh]], axis=0)
    if dw > 0:
        a = jnp.concatenate([a[:, dw:], jnp.zeros(a.shape[:1] + (dw,) + a.shape[2:], a.dtype)], axis=1)
    elif dw < 0:
        a = jnp.concatenate([jnp.zeros(a.shape[:1] + (-dw,) + a.shape[2:], a.dtype), a[:, :dw]], axis=1)
    return a


def _matmul_ct(a, w):
    return lax.dot_general(a, w, (((1,), (1,)), ((), ())),
                           preferred_element_type=jnp.float32)


def _moe_step(x_ref, rw_ref, rb_ref, eb_ref,
              w0_ref, b0_ref, dw0_ref, db0_ref,
              w1_ref, b1_ref, g1_ref,
              w2_ref, b2_ref,
              f3a_ref, f3ab_ref, f3b_ref, f3bb_ref,
              w4_ref, b4_ref, dw4_ref, db4_ref,
              w5_ref, b5_ref, g5_ref,
              w6_ref, b6_ref,
              f7a_ref, f7ab_ref, f7b_ref, f7bb_ref,
              out_ref):
    xb = x_ref[0]
    xf = xb.reshape(HW, DIM)

    gvec = jnp.mean(xf, axis=0, keepdims=True)
    logits = _matmul_ct(gvec, rw_ref[...]) + rb_ref[...]
    logits = jnp.clip(logits, -10.0, 10.0) + eb_ref[...]
    m = jnp.max(logits)
    p = jnp.exp(logits - m)
    probs = p / jnp.sum(p)
    probs = jnp.clip(probs, 1e-6, 1.0)

    iota = lax.broadcasted_iota(jnp.int32, (1, E), 1)
    v1 = jnp.max(probs)
    i1 = jnp.min(jnp.where(probs == v1, iota, E))
    sel1 = iota == i1
    rest = jnp.where(sel1, -jnp.inf, probs)
    v2 = jnp.max(rest)
    i2 = jnp.min(jnp.where((rest == v2) & (~sel1), iota, E))
    sel2 = iota == i2
    denom = v1 + v2 + 1e-8
    wa = v1 / denom
    wb = v2 / denom
    gates = jnp.where(sel1, wa, 0.0) + jnp.where(sel2, wb, 0.0)

    def gate(e):
        return jnp.sum(jnp.where(iota == e, gates, 0.0))

    g0, g1, g2, g3 = gate(0), gate(1), gate(2), gate(3)
    g4, g5, g6, g7 = gate(4), gate(5), gate(6), gate(7)

    def s_vec(fa, fab, fb, fbb):
        h = jnp.maximum(_matmul_ct(gvec, fa[...]) + fab[...], 0.0)
        return jax.nn.sigmoid(_matmul_ct(h, fb[...]) + fbb[...])

    alpha = ((wa + wb)
             + g3 * s_vec(f3a_ref, f3ab_ref, f3b_ref, f3bb_ref)
             + g7 * s_vec(f7a_ref, f7ab_ref, f7b_ref, f7bb_ref))

    btot = (g0 * b0_ref[...] + g1 * b1_ref[...] + g2 * b2_ref[...]
            + g4 * b4_ref[...] + g5 * b5_ref[...] + g6 * b6_ref[...])

    out_ref[0] = (xf * alpha + btot).reshape(H, W, DIM)

    @pl.when(g1 + g5 > 0.0)
    def _freq():
        wfr = g1 * (w1_ref[...] * g1_ref[...]) + g5 * (w5_ref[...] * g5_ref[...])
        out_ref[0] += _matmul_ct(xf, wfr).reshape(H, W, DIM)

    @pl.when(g2 + g6 > 0.0)
    def _edge():
        lap = (_shift(xb, -1, 0) + _shift(xb, 1, 0) +
               _shift(xb, 0, -1) + _shift(xb, 0, 1) - 4.0 * xb)
        wed = g2 * w2_ref[...] + g6 * w6_ref[...]
        out_ref[0] += _matmul_ct(lap.reshape(HW, DIM), wed).reshape(H, W, DIM)

    def texture(dw_ref, db_ref, w_ref, g):
        acc = jnp.broadcast_to(db_ref[...][None], (H, W, DIM))
        for a in range(3):
            for c in range(3):
                acc = acc + _shift(xb, a - 1, c - 1) * dw_ref[a * 3 + c][None, None, :]
        u = jax.nn.gelu(acc)
        out_ref[0] += _matmul_ct(u.reshape(HW, DIM), g * w_ref[...]).reshape(H, W, DIM)

    @pl.when(g0 > 0.0)
    def _tex0():
        texture(dw0_ref, db0_ref, w0_ref, g0)

    @pl.when(g4 > 0.0)
    def _tex4():
        texture(dw4_ref, db4_ref, w4_ref, g4)


def kernel(x, params):
    B = x.shape[0]
    xh = jnp.transpose(x, (0, 2, 3, 1))

    def pw(e):
        return params[f'e{e}_pw_w'].reshape(DIM, DIM)

    def row(v):
        return v[None, :]

    dw0 = params['e0_dw_w'].reshape(DIM, 9).T
    dw4 = params['e4_dw_w'].reshape(DIM, 9).T

    operands = [
        xh,
        params['router_w'], row(params['router_b']), row(params['expert_bias']),
        pw(0), row(params['e0_pw_b']), dw0, row(params['e0_dw_b']),
        pw(1), row(params['e1_pw_b']), row(params['e1_gain']),
        pw(2), row(params['e2_pw_b']),
        params['e3_fc1_w'], row(params['e3_fc1_b']),
        params['e3_fc2_w'], row(params['e3_fc2_b']),
        pw(4), row(params['e4_pw_b']), dw4, row(params['e4_dw_b']),
        pw(5), row(params['e5_pw_b']), row(params['e5_gain']),
        pw(6), row(params['e6_pw_b']),
        params['e7_fc1_w'], row(params['e7_fc1_b']),
        params['e7_fc2_w'], row(params['e7_fc2_b']),
    ]

    full = lambda a: pl.BlockSpec(a.shape, lambda b: (0,) * a.ndim)
    in_specs = [pl.BlockSpec((1, H, W, DIM), lambda b: (b, 0, 0, 0))]
    in_specs += [full(a) for a in operands[1:]]

    out_h = pl.pallas_call(
        _moe_step,
        grid=(B,),
        in_specs=in_specs,
        out_specs=pl.BlockSpec((1, H, W, DIM), lambda b: (b, 0, 0, 0)),
        out_shape=jax.ShapeDtypeStruct((B, H, W, DIM), jnp.float32),
    )(*operands)

    out = jnp.transpose(out_h, (0, 3, 1, 2))
    return (out, jnp.array(0.0, dtype=x.dtype))
